# baseline (speedup 1.0000x reference)
import jax
import jax.numpy as jnp
from jax import lax
from jax.experimental import pallas as pl
from jax.experimental.pallas import tpu as pltpu
from jax.experimental.pallas import tpu_sc as plsc

N = 10000
NPAD = 10240
EMB = 128
NC = 2
NS = 16
NW = NC * NS
CH = 128
ROWS_T = NPAD // NS
BR = 1024
GRID = NPAD // BR


def _pad1(x, n):
    return jnp.concatenate([x, jnp.zeros((n - x.shape[0],), x.dtype)])


def _padded(e):
    per = -(-e // NW)
    per = -(-per // (2 * CH)) * (2 * CH)
    return per * NW


_MESH = plsc.VectorSubcoreMesh(core_axis_name="c", subcore_axis_name="s")


def _deg_body(z1, c0, w0, c1, w1, c2, w2, dp0, dp1, dp2, cb, eb, acc):
    c = lax.axis_index("c")
    s = lax.axis_index("s")
    wid = s * NC + c
    for cols, ew, dp in ((c0, w0, dp0), (c1, w1, dp1), (c2, w2, dp2)):
        nch = cols.shape[0] // NW
        pltpu.sync_copy(z1.at[pl.ds(s * ROWS_T, ROWS_T)],
                        acc.at[pl.ds(s * ROWS_T, ROWS_T)])
        pltpu.sync_copy(cols.at[pl.ds(wid * nch, nch)], cb.at[pl.ds(0, nch)])
        pltpu.sync_copy(ew.at[pl.ds(wid * nch, nch)], eb.at[pl.ds(0, nch)])
        plsc.subcore_barrier()

        def chunk(k, _):
            pltpu.sync_copy(eb.at[k], acc.at[cb.at[k]], add=True)
            return 0
        lax.fori_loop(0, nch, chunk, 0)
        plsc.subcore_barrier()
        pltpu.sync_copy(acc.at[pl.ds(s * ROWS_T, ROWS_T)],
                        dp.at[pl.ds(c * NPAD + s * ROWS_T, ROWS_T)])
        plsc.subcore_barrier()


def _msg_body(z128, ym, yd, yp, r0, c0, w0, r1, c1, w1, r2, c2, w2,
              a0, a1, a2, riA, ciA, ewA, riB, ciB, ewB, bufA, bufB, acc,
              semIA, semIB, semA, semB):
    c = lax.axis_index("c")
    s = lax.axis_index("s")
    wid = s * NC + c
    setA = (riA, ciA, ewA, bufA, semIA, semA)
    setB = (riB, ciB, ewB, bufB, semIB, semB)
    for y, rows, cols, ew, accp in ((ym, r0, c0, w0, a0),
                                    (yd, r1, c1, w1, a1),
                                    (yp, r2, c2, w2, a2)):
        nch = rows.shape[0] // NW
        base = wid * nch
        pltpu.sync_copy(z128.at[pl.ds(s * ROWS_T, ROWS_T)],
                        acc.at[pl.ds(s * ROWS_T, ROWS_T)])
        plsc.subcore_barrier()

        def idxload(k, st):
            ri, ci, ewv, _, semI, _ = st
            pltpu.async_copy(rows.at[base + k], ri, semI)
            pltpu.async_copy(cols.at[base + k], ci, semI)
            pltpu.async_copy(ew.at[base + k], ewv, semI)

        def idxwait(k, st):
            ri, ci, ewv, _, semI, _ = st
            pltpu.make_async_copy(rows.at[base + k], ri, semI).wait()
            pltpu.make_async_copy(cols.at[base + k], ci, semI).wait()
            pltpu.make_async_copy(ew.at[base + k], ewv, semI).wait()

        def gath(st):
            ri, _, _, buf, _, semG = st
            pltpu.async_copy(y.at[ri], buf, semG)

        def gwait(st):
            ri, _, _, buf, _, semG = st
            pltpu.make_async_copy(y.at[ri], buf, semG).wait()

        def scale(st):
            _, _, ewv, buf, _, _ = st

            def egrp(g, _):
                wg = ewv[pl.ds(g * 16, 16)]
                for e16 in range(16):
                    e = g * 16 + e16
                    w = wg[e16]
                    for j in range(EMB // 16):
                        buf[e, pl.ds(j * 16, 16)] = (
                            buf[e, pl.ds(j * 16, 16)] * w)
                return 0
            lax.fori_loop(0, CH // 16, egrp, 0)

        def scat(st):
            _, ci, _, buf, _, _ = st
            pltpu.sync_copy(buf, acc.at[ci], add=True)

        idxload(0, setA)
        idxload(1, setB)
        idxwait(0, setA)
        gath(setA)
        idxwait(1, setB)
        gath(setB)

        def chunk(kk, _):
            k0 = 2 * kk
            gwait(setA)
            scale(setA)
            scat(setA)

            @pl.when(k0 + 2 < nch)
            def _():
                idxload(k0 + 2, setA)
            gwait(setB)

            @pl.when(k0 + 2 < nch)
            def _():
                idxwait(k0 + 2, setA)
                gath(setA)
            scale(setB)
            scat(setB)

            @pl.when(k0 + 3 < nch)
            def _():
                idxload(k0 + 3, setB)
                idxwait(k0 + 3, setB)
                gath(setB)
            return 0
        lax.fori_loop(0, nch // 2, chunk, 0)
        plsc.subcore_barrier()
        pltpu.sync_copy(acc.at[pl.ds(s * ROWS_T, ROWS_T)],
                        accp.at[pl.ds(c * NPAD + s * ROWS_T, ROWS_T)])
        plsc.subcore_barrier()


def _pair_body(tmwa, tm, td, tpwm, tpwd, i0, j0, i1, j1, i2, j2,
               o0, o1, o2, ib, jb, arA, brA, semA1, semA2):
    c = lax.axis_index("c")
    s = lax.axis_index("s")
    wid = s * NC + c

    def prod(a, bref):
        def pgrp(g, _):
            for e16 in range(16):
                e = g * 16 + e16
                for j in range(EMB // 16):
                    a[e, pl.ds(j * 16, 16)] = (
                        a[e, pl.ds(j * 16, 16)] * bref[e, pl.ds(j * 16, 16)])
            return 0
        lax.fori_loop(0, CH // 16, pgrp, 0)

    for ta, tb, ii, jj, out in (
            (tmwa, td, i0, j0, o0),
            (tm, tpwm, i1, j1, o1),
            (td, tpwd, i2, j2, o2)):
        ppt = ii.shape[0] // NW

        def chunk(k, _):
            b = wid * ppt + k * CH
            pltpu.sync_copy(ii.at[pl.ds(b, CH)], ib)
            pltpu.sync_copy(jj.at[pl.ds(b, CH)], jb)
            cp_a = pltpu.async_copy(ta.at[ib], arA, semA1)
            cp_b = pltpu.async_copy(tb.at[jb], brA, semA2)
            cp_a.wait()
            cp_b.wait()
            prod(arA, brA)
            pltpu.sync_copy(arA, out.at[pl.ds(b, CH)])
            return 0
        lax.fori_loop(0, ppt // CH, chunk, 0)


def _head_body(prod, w, out):
    v = jnp.sum(prod[...] * w[0][None, :], axis=1, keepdims=True)
    out[...] = 1.0 / (1.0 + jnp.exp(-(v + w[1, 0])))


def _prep_body(x0, w0, d0, x1, w1, d1, x2, w2, d2,
               y0, s0, y1, s1, y2, s2):
    for x, w, dp, y, sp in ((x0, w0, d0, y0, s0), (x1, w1, d1, y1, s1),
                            (x2, w2, d2, y2, s2)):
        deg = 1.0 + dp[0] + dp[1]
        dis = lax.rsqrt(deg)
        xw = jnp.dot(x[...], w[...], preferred_element_type=jnp.float32)
        yv = xw * dis
        y[...] = yv
        sp[...] = yv * dis


def _fin_body(a0, d0, s0, b0, a1, d1, s1, b1, a2, d2, s2, b2,
              wa, wmp, wdp, tmwa, tm, td, tpwm, tpwd):
    hids = []
    for a, dp, sp, b in ((a0, d0, s0, b0), (a1, d1, s1, b1),
                         (a2, d2, s2, b2)):
        deg = 1.0 + dp[0] + dp[1]
        dis = lax.rsqrt(deg)
        h = (a[0] + a[1]) * dis + sp[...] + b[0][None, :]
        hids.append(jnp.maximum(h, 0.0))
    hm, hd, hp = hids
    tmwa[...] = hm * wa[0][None, :]
    tm[...] = hm
    td[...] = hd
    tpwm[...] = hp * wmp[0][None, :]
    tpwd[...] = hp * wdp[0][None, :]


def kernel(mirna_emb, mirna_edgelist, mirna_edgeweight, disease_emb,
           disease_edgelist, disease_edgeweight, pcg_emb, ppi_edgelist,
           ppi_edgeweight, mirna_pcg_pairs, disease_pcg_pairs, label_tensor,
           Wm, bm, Wd, bd, Wp, bp, Wa, ba, Wmp, bmp, Wdp, bdp):
    f32 = jnp.float32
    i32 = jnp.int32

    def edges(el, ew):
        ep = _padded(el.shape[0])
        return (_pad1(el[:, 0].astype(i32), ep).reshape(-1, CH),
                _pad1(el[:, 1].astype(i32), ep).reshape(-1, CH),
                _pad1(ew, ep).reshape(-1, CH))
    rm, cm, wm_ = edges(mirna_edgelist, mirna_edgeweight)
    rd, cd, wd_ = edges(disease_edgelist, disease_edgeweight)
    rp, cp, wp_ = edges(ppi_edgelist, ppi_edgeweight)

    def pairs(pr):
        pp = _padded(pr.shape[0])
        return (_pad1(pr[:, 0].astype(i32), pp), _pad1(pr[:, 1].astype(i32), pp))
    li, lj = pairs(label_tensor)
    mi, mj = pairs(mirna_pcg_pairs)
    di, dj = pairs(disease_pcg_pairs)

    z1 = jnp.zeros((NPAD,), f32)
    z128 = jnp.zeros((NPAD, EMB), f32)
    xm = jnp.concatenate([mirna_emb, jnp.zeros((NPAD - N, EMB), f32)])
    xd = jnp.concatenate([disease_emb, jnp.zeros((NPAD - N, EMB), f32)])
    xp = jnp.concatenate([pcg_emb, jnp.zeros((NPAD - N, EMB), f32)])

    deg_k = pl.kernel(
        _deg_body, mesh=_MESH,
        out_type=[jax.ShapeDtypeStruct((NC * NPAD,), f32)] * 3,
        scratch_types=[pltpu.VMEM((80, CH), i32), pltpu.VMEM((80, CH), f32),
                       pltpu.VMEM_SHARED((NPAD,), f32)],
    )
    dpm, dpd, dpp = deg_k(z1, cm, wm_, cd, wd_, cp, wp_)
    dpm = dpm.reshape(NC, NPAD, 1)
    dpd = dpd.reshape(NC, NPAD, 1)
    dpp = dpp.reshape(NC, NPAD, 1)

    bx = pl.BlockSpec((BR, EMB), lambda i: (i, 0))
    bw = pl.BlockSpec((EMB, EMB), lambda i: (0, 0))
    bdg = pl.BlockSpec((NC, BR, 1), lambda i: (0, i, 0))
    y_m, sp_m, y_d, sp_d, y_p, sp_p = pl.pallas_call(
        _prep_body,
        grid=(GRID,),
        in_specs=[bx, bw, bdg] * 3,
        out_specs=[bx, bx] * 3,
        out_shape=[jax.ShapeDtypeStruct((NPAD, EMB), f32)] * 6,
    )(xm, Wm, dpm, xd, Wd, dpd, xp, Wp, dpp)

    msg_k = pl.kernel(
        _msg_body, mesh=_MESH,
        out_type=[jax.ShapeDtypeStruct((NC * NPAD, EMB), f32)] * 3,
        scratch_types=[pltpu.VMEM((CH,), i32), pltpu.VMEM((CH,), i32),
                       pltpu.VMEM((CH,), f32),
                       pltpu.VMEM((CH,), i32), pltpu.VMEM((CH,), i32),
                       pltpu.VMEM((CH,), f32),
                       pltpu.VMEM((CH, EMB), f32), pltpu.VMEM((CH, EMB), f32),
                       pltpu.VMEM_SHARED((NPAD, EMB), f32),
                       pltpu.SemaphoreType.DMA, pltpu.SemaphoreType.DMA,
                       pltpu.SemaphoreType.DMA, pltpu.SemaphoreType.DMA],
    )
    am, ad, ap = msg_k(z128, y_m, y_d, y_p, rm, cm, wm_, rd, cd, wd_,
                       rp, cp, wp_)
    am = am.reshape(NC, NPAD, EMB)
    ad = ad.reshape(NC, NPAD, EMB)
    ap = ap.reshape(NC, NPAD, EMB)

    ba_ = pl.BlockSpec((NC, BR, EMB), lambda i: (0, i, 0))
    bb = pl.BlockSpec((8, EMB), lambda i: (0, 0))
    bm8 = jnp.broadcast_to(bm[None, :], (8, EMB))
    bd8 = jnp.broadcast_to(bd[None, :], (8, EMB))
    bp8 = jnp.broadcast_to(bp[None, :], (8, EMB))
    wa8 = jnp.broadcast_to(Wa[:, 0][None, :], (8, EMB))
    wmp8 = jnp.broadcast_to(Wmp[:, 0][None, :], (8, EMB))
    wdp8 = jnp.broadcast_to(Wdp[:, 0][None, :], (8, EMB))
    tmwa, tm, td, tpwm, tpwd = pl.pallas_call(
        _fin_body,
        grid=(GRID,),
        in_specs=[ba_, bdg, bx, bb] * 3 + [bb] * 3,
        out_specs=[bx] * 5,
        out_shape=[jax.ShapeDtypeStruct((NPAD, EMB), f32)] * 5,
    )(am, dpm, sp_m, bm8, ad, dpd, sp_d, bd8, ap, dpp, sp_p, bp8,
      wa8, wmp8, wdp8)

    pair_k = pl.kernel(
        _pair_body, mesh=_MESH,
        out_type=[jax.ShapeDtypeStruct((li.shape[0], EMB), f32),
                  jax.ShapeDtypeStruct((mi.shape[0], EMB), f32),
                  jax.ShapeDtypeStruct((di.shape[0], EMB), f32)],
        scratch_types=[pltpu.VMEM((CH,), i32), pltpu.VMEM((CH,), i32),
                       pltpu.VMEM((CH, EMB), f32), pltpu.VMEM((CH, EMB), f32),
                       pltpu.SemaphoreType.DMA, pltpu.SemaphoreType.DMA],
    )
    pa, pmp, pdp = pair_k(tmwa, tm, td, tpwm, tpwd, li, lj, mi, mj, di, dj)

    def head(prod, wvec, bias):
        w8 = jnp.zeros((8, EMB), f32).at[0].set(wvec).at[1, 0].set(bias[0])
        np_ = prod.shape[0]
        o = pl.pallas_call(
            _head_body,
            grid=(np_ // BR,),
            in_specs=[pl.BlockSpec((BR, EMB), lambda i: (i, 0)),
                      pl.BlockSpec((8, EMB), lambda i: (0, 0))],
            out_specs=pl.BlockSpec((BR, 1), lambda i: (i, 0)),
            out_shape=jax.ShapeDtypeStruct((np_, 1), f32),
        )(prod, w8)
        return o[:, 0]

    ones = jnp.ones((EMB,), f32)
    oa = head(pa, ones, ba)
    omp = head(pmp, ones, bmp)
    odp = head(pdp, ones, bdp)
    n_lab = label_tensor.shape[0]
    n_pair = mirna_pcg_pairs.shape[0]
    return (oa[:n_lab], omp[:n_pair], odp[:n_pair])


# speedup vs baseline: 1.1005x; 1.1005x over previous
import jax
import jax.numpy as jnp
from jax import lax
from jax.experimental import pallas as pl
from jax.experimental.pallas import tpu as pltpu
from jax.experimental.pallas import tpu_sc as plsc

N = 10000
NPAD = 10240
EMB = 128
NC = 2
NS = 16
NW = NC * NS
CH = 128
ROWS_T = NPAD // NS
BR = 1024
GRID = NPAD // BR


def _pad1(x, n):
    return jnp.concatenate([x, jnp.zeros((n - x.shape[0],), x.dtype)])


def _padded(e):
    per = -(-e // NW)
    per = -(-per // (2 * CH)) * (2 * CH)
    return per * NW


_MESH = plsc.VectorSubcoreMesh(core_axis_name="c", subcore_axis_name="s")


def _deg_body(z1, c0, w0, c1, w1, c2, w2, dp0, dp1, dp2, cb, eb, acc):
    c = lax.axis_index("c")
    s = lax.axis_index("s")
    wid = s * NC + c
    for cols, ew, dp in ((c0, w0, dp0), (c1, w1, dp1), (c2, w2, dp2)):
        nch = cols.shape[0] // NW
        pltpu.sync_copy(z1.at[pl.ds(s * ROWS_T, ROWS_T)],
                        acc.at[pl.ds(s * ROWS_T, ROWS_T)])
        pltpu.sync_copy(cols.at[pl.ds(wid * nch, nch)], cb.at[pl.ds(0, nch)])
        pltpu.sync_copy(ew.at[pl.ds(wid * nch, nch)], eb.at[pl.ds(0, nch)])
        plsc.subcore_barrier()

        def chunk(k, _):
            pltpu.sync_copy(eb.at[k], acc.at[cb.at[k]], add=True)
            return 0
        lax.fori_loop(0, nch, chunk, 0)
        plsc.subcore_barrier()
        pltpu.sync_copy(acc.at[pl.ds(s * ROWS_T, ROWS_T)],
                        dp.at[pl.ds(c * NPAD + s * ROWS_T, ROWS_T)])
        plsc.subcore_barrier()


def _msg_body(z128, ym, yd, yp, r0, c0, w0, r1, c1, w1, r2, c2, w2,
              a0, a1, a2, riA, ciA, ewA, riB, ciB, ewB, bufA, bufB, acc,
              semIA, semIB, semA, semB):
    c = lax.axis_index("c")
    s = lax.axis_index("s")
    wid = s * NC + c
    setA = (riA, ciA, ewA, bufA, semIA, semA)
    setB = (riB, ciB, ewB, bufB, semIB, semB)
    for y, rows, cols, ew, accp in ((ym, r0, c0, w0, a0),
                                    (yd, r1, c1, w1, a1),
                                    (yp, r2, c2, w2, a2)):
        nch = rows.shape[0] // NW
        base = wid * nch
        pltpu.sync_copy(z128.at[pl.ds(s * ROWS_T, ROWS_T)],
                        acc.at[pl.ds(s * ROWS_T, ROWS_T)])
        plsc.subcore_barrier()

        def idxload(k, st):
            ri, ci, ewv, _, semI, _ = st
            pltpu.async_copy(rows.at[base + k], ri, semI)
            pltpu.async_copy(cols.at[base + k], ci, semI)
            pltpu.async_copy(ew.at[base + k], ewv, semI)

        def idxwait(k, st):
            ri, ci, ewv, _, semI, _ = st
            pltpu.make_async_copy(rows.at[base + k], ri, semI).wait()
            pltpu.make_async_copy(cols.at[base + k], ci, semI).wait()
            pltpu.make_async_copy(ew.at[base + k], ewv, semI).wait()

        def gath(st):
            ri, _, _, buf, _, semG = st
            pltpu.async_copy(y.at[ri], buf, semG)

        def gwait(st):
            ri, _, _, buf, _, semG = st
            pltpu.make_async_copy(y.at[ri], buf, semG).wait()

        def scale(st):
            _, _, ewv, buf, _, _ = st

            def egrp(g, _):
                wg = ewv[pl.ds(g * 16, 16)]
                for e16 in range(16):
                    e = g * 16 + e16
                    w = wg[e16]
                    for j in range(EMB // 16):
                        buf[e, pl.ds(j * 16, 16)] = (
                            buf[e, pl.ds(j * 16, 16)] * w)
                return 0
            lax.fori_loop(0, CH // 16, egrp, 0)

        def scat(st):
            _, ci, _, buf, _, _ = st
            pltpu.sync_copy(buf, acc.at[ci], add=True)

        idxload(0, setA)
        idxload(1, setB)
        idxwait(0, setA)
        gath(setA)
        idxwait(1, setB)
        gath(setB)

        def chunk(kk, _):
            k0 = 2 * kk
            gwait(setA)
            scale(setA)
            scat(setA)

            @pl.when(k0 + 2 < nch)
            def _():
                idxload(k0 + 2, setA)
            gwait(setB)

            @pl.when(k0 + 2 < nch)
            def _():
                idxwait(k0 + 2, setA)
                gath(setA)
            scale(setB)
            scat(setB)

            @pl.when(k0 + 3 < nch)
            def _():
                idxload(k0 + 3, setB)
                idxwait(k0 + 3, setB)
                gath(setB)
            return 0
        lax.fori_loop(0, nch // 2, chunk, 0)
        plsc.subcore_barrier()
        pltpu.sync_copy(acc.at[pl.ds(s * ROWS_T, ROWS_T)],
                        accp.at[pl.ds(c * NPAD + s * ROWS_T, ROWS_T)])
        plsc.subcore_barrier()


def _pair_body(tmwa, tm, td, tpwm, tpwd, i0, j0, i1, j1, i2, j2,
               o0, o1, o2, ib, jb, arA, brA, obuf, semA1, semA2):
    c = lax.axis_index("c")
    s = lax.axis_index("s")
    wid = s * NC + c

    def prod(a, bref, ob):
        def pgrp(g, _):
            for e16 in range(16):
                e = g * 16 + e16
                acc = a[e, pl.ds(0, 16)] * bref[e, pl.ds(0, 16)]
                for j in range(1, EMB // 16):
                    acc = acc + (a[e, pl.ds(j * 16, 16)]
                                 * bref[e, pl.ds(j * 16, 16)])
                ob[e, pl.ds(0, 16)] = acc
            return 0
        lax.fori_loop(0, CH // 16, pgrp, 0)

    for ta, tb, ii, jj, out in (
            (tmwa, td, i0, j0, o0),
            (tm, tpwm, i1, j1, o1),
            (td, tpwd, i2, j2, o2)):
        ppt = ii.shape[0] // NW

        def chunk(k, _):
            b = wid * ppt + k * CH
            pltpu.sync_copy(ii.at[pl.ds(b, CH)], ib)
            pltpu.sync_copy(jj.at[pl.ds(b, CH)], jb)
            cp_a = pltpu.async_copy(ta.at[ib], arA, semA1)
            cp_b = pltpu.async_copy(tb.at[jb], brA, semA2)
            cp_a.wait()
            cp_b.wait()
            prod(arA, brA, obuf)
            pltpu.sync_copy(obuf, out.at[pl.ds(b, CH)])
            return 0
        lax.fori_loop(0, ppt // CH, chunk, 0)


def _head_body(prod, w, out):
    v = jnp.sum(prod[...], axis=1, keepdims=True)
    out[...] = 1.0 / (1.0 + jnp.exp(-(v + w[1, 0])))


def _prep_body(x0, w0, d0, x1, w1, d1, x2, w2, d2,
               y0, s0, y1, s1, y2, s2):
    for x, w, dp, y, sp in ((x0, w0, d0, y0, s0), (x1, w1, d1, y1, s1),
                            (x2, w2, d2, y2, s2)):
        deg = 1.0 + dp[0] + dp[1]
        dis = lax.rsqrt(deg)
        xw = jnp.dot(x[...], w[...], preferred_element_type=jnp.float32)
        yv = xw * dis
        y[...] = yv
        sp[...] = yv * dis


def _fin_body(a0, d0, s0, b0, a1, d1, s1, b1, a2, d2, s2, b2,
              wa, wmp, wdp, tmwa, tm, td, tpwm, tpwd):
    hids = []
    for a, dp, sp, b in ((a0, d0, s0, b0), (a1, d1, s1, b1),
                         (a2, d2, s2, b2)):
        deg = 1.0 + dp[0] + dp[1]
        dis = lax.rsqrt(deg)
        h = (a[0] + a[1]) * dis + sp[...] + b[0][None, :]
        hids.append(jnp.maximum(h, 0.0))
    hm, hd, hp = hids
    tmwa[...] = hm * wa[0][None, :]
    tm[...] = hm
    td[...] = hd
    tpwm[...] = hp * wmp[0][None, :]
    tpwd[...] = hp * wdp[0][None, :]


def kernel(mirna_emb, mirna_edgelist, mirna_edgeweight, disease_emb,
           disease_edgelist, disease_edgeweight, pcg_emb, ppi_edgelist,
           ppi_edgeweight, mirna_pcg_pairs, disease_pcg_pairs, label_tensor,
           Wm, bm, Wd, bd, Wp, bp, Wa, ba, Wmp, bmp, Wdp, bdp):
    f32 = jnp.float32
    i32 = jnp.int32

    def edges(el, ew):
        ep = _padded(el.shape[0])
        return (_pad1(el[:, 0].astype(i32), ep).reshape(-1, CH),
                _pad1(el[:, 1].astype(i32), ep).reshape(-1, CH),
                _pad1(ew, ep).reshape(-1, CH))
    rm, cm, wm_ = edges(mirna_edgelist, mirna_edgeweight)
    rd, cd, wd_ = edges(disease_edgelist, disease_edgeweight)
    rp, cp, wp_ = edges(ppi_edgelist, ppi_edgeweight)

    def pairs(pr):
        pp = _padded(pr.shape[0])
        return (_pad1(pr[:, 0].astype(i32), pp), _pad1(pr[:, 1].astype(i32), pp))
    li, lj = pairs(label_tensor)
    mi, mj = pairs(mirna_pcg_pairs)
    di, dj = pairs(disease_pcg_pairs)

    z1 = jnp.zeros((NPAD,), f32)
    z128 = jnp.zeros((NPAD, EMB), f32)
    xm = jnp.concatenate([mirna_emb, jnp.zeros((NPAD - N, EMB), f32)])
    xd = jnp.concatenate([disease_emb, jnp.zeros((NPAD - N, EMB), f32)])
    xp = jnp.concatenate([pcg_emb, jnp.zeros((NPAD - N, EMB), f32)])

    deg_k = pl.kernel(
        _deg_body, mesh=_MESH,
        out_type=[jax.ShapeDtypeStruct((NC * NPAD,), f32)] * 3,
        scratch_types=[pltpu.VMEM((80, CH), i32), pltpu.VMEM((80, CH), f32),
                       pltpu.VMEM_SHARED((NPAD,), f32)],
    )
    dpm, dpd, dpp = deg_k(z1, cm, wm_, cd, wd_, cp, wp_)
    dpm = dpm.reshape(NC, NPAD, 1)
    dpd = dpd.reshape(NC, NPAD, 1)
    dpp = dpp.reshape(NC, NPAD, 1)

    bx = pl.BlockSpec((BR, EMB), lambda i: (i, 0))
    bw = pl.BlockSpec((EMB, EMB), lambda i: (0, 0))
    bdg = pl.BlockSpec((NC, BR, 1), lambda i: (0, i, 0))
    y_m, sp_m, y_d, sp_d, y_p, sp_p = pl.pallas_call(
        _prep_body,
        grid=(GRID,),
        in_specs=[bx, bw, bdg] * 3,
        out_specs=[bx, bx] * 3,
        out_shape=[jax.ShapeDtypeStruct((NPAD, EMB), f32)] * 6,
    )(xm, Wm, dpm, xd, Wd, dpd, xp, Wp, dpp)

    msg_k = pl.kernel(
        _msg_body, mesh=_MESH,
        out_type=[jax.ShapeDtypeStruct((NC * NPAD, EMB), f32)] * 3,
        scratch_types=[pltpu.VMEM((CH,), i32), pltpu.VMEM((CH,), i32),
                       pltpu.VMEM((CH,), f32),
                       pltpu.VMEM((CH,), i32), pltpu.VMEM((CH,), i32),
                       pltpu.VMEM((CH,), f32),
                       pltpu.VMEM((CH, EMB), f32), pltpu.VMEM((CH, EMB), f32),
                       pltpu.VMEM_SHARED((NPAD, EMB), f32),
                       pltpu.SemaphoreType.DMA, pltpu.SemaphoreType.DMA,
                       pltpu.SemaphoreType.DMA, pltpu.SemaphoreType.DMA],
    )
    am, ad, ap = msg_k(z128, y_m, y_d, y_p, rm, cm, wm_, rd, cd, wd_,
                       rp, cp, wp_)
    am = am.reshape(NC, NPAD, EMB)
    ad = ad.reshape(NC, NPAD, EMB)
    ap = ap.reshape(NC, NPAD, EMB)

    ba_ = pl.BlockSpec((NC, BR, EMB), lambda i: (0, i, 0))
    bb = pl.BlockSpec((8, EMB), lambda i: (0, 0))
    bm8 = jnp.broadcast_to(bm[None, :], (8, EMB))
    bd8 = jnp.broadcast_to(bd[None, :], (8, EMB))
    bp8 = jnp.broadcast_to(bp[None, :], (8, EMB))
    wa8 = jnp.broadcast_to(Wa[:, 0][None, :], (8, EMB))
    wmp8 = jnp.broadcast_to(Wmp[:, 0][None, :], (8, EMB))
    wdp8 = jnp.broadcast_to(Wdp[:, 0][None, :], (8, EMB))
    tmwa, tm, td, tpwm, tpwd = pl.pallas_call(
        _fin_body,
        grid=(GRID,),
        in_specs=[ba_, bdg, bx, bb] * 3 + [bb] * 3,
        out_specs=[bx] * 5,
        out_shape=[jax.ShapeDtypeStruct((NPAD, EMB), f32)] * 5,
    )(am, dpm, sp_m, bm8, ad, dpd, sp_d, bd8, ap, dpp, sp_p, bp8,
      wa8, wmp8, wdp8)

    pair_k = pl.kernel(
        _pair_body, mesh=_MESH,
        out_type=[jax.ShapeDtypeStruct((li.shape[0], 16), f32),
                  jax.ShapeDtypeStruct((mi.shape[0], 16), f32),
                  jax.ShapeDtypeStruct((di.shape[0], 16), f32)],
        scratch_types=[pltpu.VMEM((CH,), i32), pltpu.VMEM((CH,), i32),
                       pltpu.VMEM((CH, EMB), f32), pltpu.VMEM((CH, EMB), f32),
                       pltpu.VMEM((CH, 16), f32),
                       pltpu.SemaphoreType.DMA, pltpu.SemaphoreType.DMA],
    )
    pa, pmp, pdp = pair_k(tmwa, tm, td, tpwm, tpwd, li, lj, mi, mj, di, dj)

    def head(prod, bias):
        w8 = jnp.zeros((8, EMB), f32).at[1, 0].set(bias[0])
        np_ = prod.shape[0]
        o = pl.pallas_call(
            _head_body,
            grid=(np_ // BR,),
            in_specs=[pl.BlockSpec((BR, 16), lambda i: (i, 0)),
                      pl.BlockSpec((8, EMB), lambda i: (0, 0))],
            out_specs=pl.BlockSpec((BR, 1), lambda i: (i, 0)),
            out_shape=jax.ShapeDtypeStruct((np_, 1), f32),
        )(prod, w8)
        return o[:, 0]

    oa = head(pa, ba)
    omp = head(pmp, bmp)
    odp = head(pdp, bdp)
    n_lab = label_tensor.shape[0]
    n_pair = mirna_pcg_pairs.shape[0]
    return (oa[:n_lab], omp[:n_pair], odp[:n_pair])


# speedup vs baseline: 1.2356x; 1.1228x over previous
import jax
import jax.numpy as jnp
from jax import lax
from jax.experimental import pallas as pl
from jax.experimental.pallas import tpu as pltpu
from jax.experimental.pallas import tpu_sc as plsc

N = 10000
NPAD = 10240
EMB = 128
NC = 2
NS = 16
NW = NC * NS
CH = 128
ROWS_T = NPAD // NS
BR = 1024
GRID = NPAD // BR


def _pad1(x, n):
    return jnp.concatenate([x, jnp.zeros((n - x.shape[0],), x.dtype)])


def _padded(e):
    per = -(-e // NW)
    per = -(-per // (2 * CH)) * (2 * CH)
    return per * NW


_MESH = plsc.VectorSubcoreMesh(core_axis_name="c", subcore_axis_name="s")


def _deg_body(z1, c0, w0, c1, w1, c2, w2, dp0, dp1, dp2, cb, eb, acc):
    c = lax.axis_index("c")
    s = lax.axis_index("s")
    wid = s * NC + c
    for cols, ew, dp in ((c0, w0, dp0), (c1, w1, dp1), (c2, w2, dp2)):
        nch = cols.shape[0] // NW
        pltpu.sync_copy(z1.at[pl.ds(s * ROWS_T, ROWS_T)],
                        acc.at[pl.ds(s * ROWS_T, ROWS_T)])
        pltpu.sync_copy(cols.at[pl.ds(wid * nch, nch)], cb.at[pl.ds(0, nch)])
        pltpu.sync_copy(ew.at[pl.ds(wid * nch, nch)], eb.at[pl.ds(0, nch)])
        plsc.subcore_barrier()

        def chunk(k, _):
            pltpu.sync_copy(eb.at[k], acc.at[cb.at[k]], add=True)
            return 0
        lax.fori_loop(0, nch, chunk, 0)
        plsc.subcore_barrier()
        pltpu.sync_copy(acc.at[pl.ds(s * ROWS_T, ROWS_T)],
                        dp.at[pl.ds(c * NPAD + s * ROWS_T, ROWS_T)])
        plsc.subcore_barrier()


def _msg_body(z128, ym, yd, yp, r0, c0, w0, r1, c1, w1, r2, c2, w2,
              a0, a1, a2, riA, ciA, ewA, riB, ciB, ewB, bufA, bufB, acc,
              semIA, semIB, semA, semB):
    c = lax.axis_index("c")
    s = lax.axis_index("s")
    wid = s * NC + c
    setA = (riA, ciA, ewA, bufA, semIA, semA)
    setB = (riB, ciB, ewB, bufB, semIB, semB)
    for y, rows, cols, ew, accp in ((ym, r0, c0, w0, a0),
                                    (yd, r1, c1, w1, a1),
                                    (yp, r2, c2, w2, a2)):
        nch = rows.shape[0] // NW
        base = wid * nch
        pltpu.sync_copy(z128.at[pl.ds(s * ROWS_T, ROWS_T)],
                        acc.at[pl.ds(s * ROWS_T, ROWS_T)])
        plsc.subcore_barrier()

        def idxload(k, st):
            ri, ci, ewv, _, semI, _ = st
            pltpu.async_copy(rows.at[base + k], ri, semI)
            pltpu.async_copy(cols.at[base + k], ci, semI)
            pltpu.async_copy(ew.at[base + k], ewv, semI)

        def idxwait(k, st):
            ri, ci, ewv, _, semI, _ = st
            pltpu.make_async_copy(rows.at[base + k], ri, semI).wait()
            pltpu.make_async_copy(cols.at[base + k], ci, semI).wait()
            pltpu.make_async_copy(ew.at[base + k], ewv, semI).wait()

        def gath(st):
            ri, _, _, buf, _, semG = st
            pltpu.async_copy(y.at[ri], buf, semG)

        def gwait(st):
            ri, _, _, buf, _, semG = st
            pltpu.make_async_copy(y.at[ri], buf, semG).wait()

        def scale(st):
            _, _, ewv, buf, _, _ = st

            def egrp(g, _):
                wg = ewv[pl.ds(g * 16, 16)]
                for e16 in range(16):
                    e = g * 16 + e16
                    w = wg[e16]
                    for j in range(EMB // 16):
                        buf[e, pl.ds(j * 16, 16)] = (
                            buf[e, pl.ds(j * 16, 16)] * w)
                return 0
            lax.fori_loop(0, CH // 16, egrp, 0)

        def scat(st):
            _, ci, _, buf, _, _ = st
            pltpu.sync_copy(buf, acc.at[ci], add=True)

        idxload(0, setA)
        idxload(1, setB)
        idxwait(0, setA)
        gath(setA)
        idxwait(1, setB)
        gath(setB)

        def chunk(kk, _):
            k0 = 2 * kk
            gwait(setA)
            scale(setA)
            scat(setA)

            @pl.when(k0 + 2 < nch)
            def _():
                idxload(k0 + 2, setA)
            gwait(setB)

            @pl.when(k0 + 2 < nch)
            def _():
                idxwait(k0 + 2, setA)
                gath(setA)
            scale(setB)
            scat(setB)

            @pl.when(k0 + 3 < nch)
            def _():
                idxload(k0 + 3, setB)
                idxwait(k0 + 3, setB)
                gath(setB)
            return 0
        lax.fori_loop(0, nch // 2, chunk, 0)
        plsc.subcore_barrier()
        pltpu.sync_copy(acc.at[pl.ds(s * ROWS_T, ROWS_T)],
                        accp.at[pl.ds(c * NPAD + s * ROWS_T, ROWS_T)])
        plsc.subcore_barrier()


CH5 = 128
TROWS = 10112


def _pair_body(tmwa, tm, td, tpwm, tpwd, i0, j0, i1, j1, i2, j2,
               o0, o1, o2,
               ib0, jb0, ar0, br0, ib1, jb1, ar1, br1,
               obuf, sa0, sb0, sa1, sb1):
    c = lax.axis_index("c")
    s = lax.axis_index("s")
    wid = s * NC + c
    st = ((ib0, jb0, ar0, br0, sa0, sb0),
          (ib1, jb1, ar1, br1, sa1, sb1))
    def prod(a, bref, ob):
        def pgrp(g, _):
            for e16 in range(16):
                e = g * 16 + e16
                acc = a[e, pl.ds(0, 16)] * bref[e, pl.ds(0, 16)]
                for j in range(1, EMB // 16):
                    acc = acc + (a[e, pl.ds(j * 16, 16)]
                                 * bref[e, pl.ds(j * 16, 16)])
                ob[e, pl.ds(0, 16)] = acc
            return 0
        lax.fori_loop(0, CH5 // 16, pgrp, 0)

    for ta, tb, ii, jj, out in (
            (tmwa, td, i0, j0, o0),
            (tm, tpwm, i1, j1, o1),
            (td, tpwd, i2, j2, o2)):
        ppt = ii.shape[0] // NW
        nch = ppt // CH5
        base = wid * ppt

        def fire(k, ss):
            ib, jb, ar, br, sa, sb = ss
            pltpu.sync_copy(ii.at[pl.ds(base + k * CH5, CH5)], ib)
            pltpu.sync_copy(jj.at[pl.ds(base + k * CH5, CH5)], jb)
            pltpu.async_copy(ta.at[ib], ar, sa)
            pltpu.async_copy(tb.at[jb], br, sb)

        def wtp(k, ss):
            ib, jb, ar, br, sa, sb = ss
            pltpu.make_async_copy(ta.at[ib], ar, sa).wait()
            pltpu.make_async_copy(tb.at[jb], br, sb).wait()
            prod(ar, br, obuf)
            pltpu.sync_copy(obuf, out.at[pl.ds(base + k * CH5, CH5)])

        fire(0, st[0])
        fire(1, st[1])

        def chunk(kk, _):
            k0 = 2 * kk
            wtp(k0, st[0])

            @pl.when(k0 + 2 < nch)
            def _():
                fire(k0 + 2, st[0])
            wtp(k0 + 1, st[1])

            @pl.when(k0 + 3 < nch)
            def _():
                fire(k0 + 3, st[1])
            return 0
        lax.fori_loop(0, nch // 2, chunk, 0)
        plsc.subcore_barrier()


def _head_body(prod, w, out):
    v = jnp.sum(prod[...], axis=1, keepdims=True)
    out[...] = 1.0 / (1.0 + jnp.exp(-(v + w[1, 0])))


def _prep_body(x0, w0, d0, x1, w1, d1, x2, w2, d2,
               y0, s0, y1, s1, y2, s2):
    for x, w, dp, y, sp in ((x0, w0, d0, y0, s0), (x1, w1, d1, y1, s1),
                            (x2, w2, d2, y2, s2)):
        deg = 1.0 + dp[0] + dp[1]
        dis = lax.rsqrt(deg)
        xw = jnp.dot(x[...], w[...], preferred_element_type=jnp.float32)
        yv = xw * dis
        y[...] = yv
        sp[...] = yv * dis


def _fin_body(a0, d0, s0, b0, a1, d1, s1, b1, a2, d2, s2, b2,
              wa, wmp, wdp, tmwa, tm, td, tpwm, tpwd):
    hids = []
    for a, dp, sp, b in ((a0, d0, s0, b0), (a1, d1, s1, b1),
                         (a2, d2, s2, b2)):
        deg = 1.0 + dp[0] + dp[1]
        dis = lax.rsqrt(deg)
        h = (a[0] + a[1]) * dis + sp[...] + b[0][None, :]
        hids.append(jnp.maximum(h, 0.0))
    hm, hd, hp = hids
    tmwa[...] = hm * wa[0][None, :]
    tm[...] = hm
    td[...] = hd
    tpwm[...] = hp * wmp[0][None, :]
    tpwd[...] = hp * wdp[0][None, :]


def kernel(mirna_emb, mirna_edgelist, mirna_edgeweight, disease_emb,
           disease_edgelist, disease_edgeweight, pcg_emb, ppi_edgelist,
           ppi_edgeweight, mirna_pcg_pairs, disease_pcg_pairs, label_tensor,
           Wm, bm, Wd, bd, Wp, bp, Wa, ba, Wmp, bmp, Wdp, bdp):
    f32 = jnp.float32
    i32 = jnp.int32

    def edges(el, ew):
        ep = _padded(el.shape[0])
        return (_pad1(el[:, 0].astype(i32), ep).reshape(-1, CH),
                _pad1(el[:, 1].astype(i32), ep).reshape(-1, CH),
                _pad1(ew, ep).reshape(-1, CH))
    rm, cm, wm_ = edges(mirna_edgelist, mirna_edgeweight)
    rd, cd, wd_ = edges(disease_edgelist, disease_edgeweight)
    rp, cp, wp_ = edges(ppi_edgelist, ppi_edgeweight)

    def pairs(pr):
        pp = _padded(pr.shape[0])
        return (_pad1(pr[:, 0].astype(i32), pp), _pad1(pr[:, 1].astype(i32), pp))
    li, lj = pairs(label_tensor)
    mi, mj = pairs(mirna_pcg_pairs)
    di, dj = pairs(disease_pcg_pairs)

    z1 = jnp.zeros((NPAD,), f32)
    z128 = jnp.zeros((NPAD, EMB), f32)
    xm = jnp.concatenate([mirna_emb, jnp.zeros((NPAD - N, EMB), f32)])
    xd = jnp.concatenate([disease_emb, jnp.zeros((NPAD - N, EMB), f32)])
    xp = jnp.concatenate([pcg_emb, jnp.zeros((NPAD - N, EMB), f32)])

    deg_k = pl.kernel(
        _deg_body, mesh=_MESH,
        out_type=[jax.ShapeDtypeStruct((NC * NPAD,), f32)] * 3,
        scratch_types=[pltpu.VMEM((80, CH), i32), pltpu.VMEM((80, CH), f32),
                       pltpu.VMEM_SHARED((NPAD,), f32)],
    )
    dpm, dpd, dpp = deg_k(z1, cm, wm_, cd, wd_, cp, wp_)
    dpm = dpm.reshape(NC, NPAD, 1)
    dpd = dpd.reshape(NC, NPAD, 1)
    dpp = dpp.reshape(NC, NPAD, 1)

    bx = pl.BlockSpec((BR, EMB), lambda i: (i, 0))
    bw = pl.BlockSpec((EMB, EMB), lambda i: (0, 0))
    bdg = pl.BlockSpec((NC, BR, 1), lambda i: (0, i, 0))
    y_m, sp_m, y_d, sp_d, y_p, sp_p = pl.pallas_call(
        _prep_body,
        grid=(GRID,),
        in_specs=[bx, bw, bdg] * 3,
        out_specs=[bx, bx] * 3,
        out_shape=[jax.ShapeDtypeStruct((NPAD, EMB), f32)] * 6,
    )(xm, Wm, dpm, xd, Wd, dpd, xp, Wp, dpp)

    msg_k = pl.kernel(
        _msg_body, mesh=_MESH,
        out_type=[jax.ShapeDtypeStruct((NC * NPAD, EMB), f32)] * 3,
        scratch_types=[pltpu.VMEM((CH,), i32), pltpu.VMEM((CH,), i32),
                       pltpu.VMEM((CH,), f32),
                       pltpu.VMEM((CH,), i32), pltpu.VMEM((CH,), i32),
                       pltpu.VMEM((CH,), f32),
                       pltpu.VMEM((CH, EMB), f32), pltpu.VMEM((CH, EMB), f32),
                       pltpu.VMEM_SHARED((NPAD, EMB), f32),
                       pltpu.SemaphoreType.DMA, pltpu.SemaphoreType.DMA,
                       pltpu.SemaphoreType.DMA, pltpu.SemaphoreType.DMA],
    )
    am, ad, ap = msg_k(z128, y_m, y_d, y_p, rm, cm, wm_, rd, cd, wd_,
                       rp, cp, wp_)
    am = am.reshape(NC, NPAD, EMB)
    ad = ad.reshape(NC, NPAD, EMB)
    ap = ap.reshape(NC, NPAD, EMB)

    ba_ = pl.BlockSpec((NC, BR, EMB), lambda i: (0, i, 0))
    bb = pl.BlockSpec((8, EMB), lambda i: (0, 0))
    bm8 = jnp.broadcast_to(bm[None, :], (8, EMB))
    bd8 = jnp.broadcast_to(bd[None, :], (8, EMB))
    bp8 = jnp.broadcast_to(bp[None, :], (8, EMB))
    wa8 = jnp.broadcast_to(Wa[:, 0][None, :], (8, EMB))
    wmp8 = jnp.broadcast_to(Wmp[:, 0][None, :], (8, EMB))
    wdp8 = jnp.broadcast_to(Wdp[:, 0][None, :], (8, EMB))
    tmwa, tm, td, tpwm, tpwd = pl.pallas_call(
        _fin_body,
        grid=(GRID,),
        in_specs=[ba_, bdg, bx, bb] * 3 + [bb] * 3,
        out_specs=[bx] * 5,
        out_shape=[jax.ShapeDtypeStruct((NPAD, EMB), f32)] * 5,
    )(am, dpm, sp_m, bm8, ad, dpd, sp_d, bd8, ap, dpp, sp_p, bp8,
      wa8, wmp8, wdp8)

    pair_k = pl.kernel(
        _pair_body, mesh=_MESH,
        out_type=[jax.ShapeDtypeStruct((li.shape[0], 16), f32),
                  jax.ShapeDtypeStruct((mi.shape[0], 16), f32),
                  jax.ShapeDtypeStruct((di.shape[0], 16), f32)],
        scratch_types=([pltpu.VMEM((CH5,), i32), pltpu.VMEM((CH5,), i32),
                        pltpu.VMEM((CH5, EMB), f32),
                        pltpu.VMEM((CH5, EMB), f32)] * 2
                       + [pltpu.VMEM((CH5, 16), f32)]
                       + [pltpu.SemaphoreType.DMA] * 4),
    )
    pa, pmp, pdp = pair_k(tmwa, tm, td, tpwm, tpwd, li, lj, mi, mj, di, dj)

    def head(prod, bias):
        w8 = jnp.zeros((8, EMB), f32).at[1, 0].set(bias[0])
        np_ = prod.shape[0]
        o = pl.pallas_call(
            _head_body,
            grid=(np_ // BR,),
            in_specs=[pl.BlockSpec((BR, 16), lambda i: (i, 0)),
                      pl.BlockSpec((8, EMB), lambda i: (0, 0))],
            out_specs=pl.BlockSpec((BR, 1), lambda i: (i, 0)),
            out_shape=jax.ShapeDtypeStruct((np_, 1), f32),
        )(prod, w8)
        return o[:, 0]

    oa = head(pa, ba)
    omp = head(pmp, bmp)
    odp = head(pdp, bdp)
    n_lab = label_tensor.shape[0]
    n_pair = mirna_pcg_pairs.shape[0]
    return (oa[:n_lab], omp[:n_pair], odp[:n_pair])
